# baseline (speedup 1.0000x reference)
import functools

import jax
import jax.numpy as jnp
from jax import lax
from jax.experimental import pallas as pl
from jax.experimental.pallas import tpu as pltpu
from jax.experimental.pallas import tpu_sc as plsc

N_SLOTS = 100000
D_STATE = 8
NGRAM = 16
N_BLADES = 8
ROW = N_BLADES * D_STATE

NC = 2
NS = 16
L = 16
NW = NC * NS

B_WIN = 16
S_WIN = 4096
TOKENS = B_WIN * S_WIN
TOK_PER_W = TOKENS // NW
SUB = 512
NSUB = TOK_PER_W // SUB
GROUPS = SUB // L
W_PER_B = S_WIN // TOK_PER_W

FNV_INIT = -2128831035
FNV_PRIME = 16777619
TWO32_F = 4294967296.0
INV_SLOTS = 1.0 / float(N_SLOTS)


def _hash_group(bw_v, base):
    h = jnp.full((L,), FNV_INIT, dtype=jnp.int32)
    for i in range(NGRAM):
        b = bw_v.at[i][pl.ds(base, L)]
        h = (h ^ b) * FNV_PRIME
    uf = h.astype(jnp.float32) + jnp.where(h < 0, TWO32_F, 0.0)
    q = (uf * INV_SLOTS).astype(jnp.int32)
    r = h - q * N_SLOTS
    r = jnp.where(r < 0, r + N_SLOTS, r)
    r = jnp.where(r >= N_SLOTS, r - N_SLOTS, r)
    return r


def _body(bw_hbm, bank_hbm, out_hbm, bw_v, addr_v, rows_v, stage_v, sem):
    wid = lax.axis_index("s") * NC + lax.axis_index("c")
    wb = wid // W_PER_B
    ws = (wid % W_PER_B) * TOK_PER_W
    pltpu.sync_copy(bw_hbm.at[wb, :, pl.ds(ws, TOK_PER_W)], bw_v)
    iota = lax.iota(jnp.int32, L)

    for sub in range(NSUB):
        def group_body(g, _, sub=sub):
            addr = _hash_group(bw_v, sub * SUB + g * L)
            addr_v[pl.ds(g * L, L)] = addr
            return 0

        lax.fori_loop(0, GROUPS, group_body, 0)
        pltpu.async_copy(bank_hbm.at[addr_v], rows_v, sem).wait()

        def col_body(c, _):
            col = jnp.full((L,), c, dtype=jnp.int32)
            jv = col // D_STATE
            dv = col % D_STATE
            for g2 in range(GROUPS):
                rows16 = plsc.load_gather(rows_v, [g2 * L + iota, col])
                plsc.store_scatter(stage_v, [jv, dv, g2 * L + iota], rows16)
            return 0

        lax.fori_loop(0, ROW, col_body, 0)
        s0 = ws + sub * SUB
        pltpu.sync_copy(stage_v, out_hbm.at[wb, :, :, pl.ds(s0, SUB)])


@jax.jit
def _sc_gather(bw_t, bank64):
    mesh = plsc.VectorSubcoreMesh(
        core_axis_name="c", subcore_axis_name="s", num_cores=NC, num_subcores=NS
    )
    return pl.kernel(
        _body,
        out_type=jax.ShapeDtypeStruct(
            (B_WIN, N_BLADES, D_STATE, S_WIN), jnp.float32
        ),
        mesh=mesh,
        scratch_types=[
            pltpu.VMEM((NGRAM, TOK_PER_W), jnp.int32),
            pltpu.VMEM((SUB,), jnp.int32),
            pltpu.VMEM((SUB, ROW), jnp.float32),
            pltpu.VMEM((N_BLADES, D_STATE, SUB), jnp.float32),
            pltpu.SemaphoreType.DMA,
        ],
        compiler_params=pltpu.CompilerParams(
            needs_layout_passes=False, use_tc_tiling_on_sc=False
        ),
    )(bw_t, bank64)


def kernel(byte_window, bank):
    bw_t = jnp.transpose(byte_window, (0, 2, 1))
    bank64 = jnp.transpose(bank, (1, 0, 2)).reshape(N_SLOTS, ROW)
    out_t = _sc_gather(bw_t, bank64)
    return jnp.transpose(out_t, (0, 3, 1, 2))


# speedup vs baseline: 1.0557x; 1.0557x over previous
import functools

import jax
import jax.numpy as jnp
from jax import lax
from jax.experimental import pallas as pl
from jax.experimental.pallas import tpu as pltpu
from jax.experimental.pallas import tpu_sc as plsc

N_SLOTS = 100000
D_STATE = 8
NGRAM = 16
N_BLADES = 8
ROW = N_BLADES * D_STATE

NC = 2
NS = 16
L = 16
NW = NC * NS

B_WIN = 16
S_WIN = 4096
TOKENS = B_WIN * S_WIN
TOK_PER_W = TOKENS // NW
SUB = 256
NSUB = TOK_PER_W // SUB
GROUPS = SUB // L
W_PER_B = S_WIN // TOK_PER_W

FNV_INIT = -2128831035
FNV_PRIME = 16777619
TWO32_F = 4294967296.0
INV_SLOTS = 1.0 / float(N_SLOTS)


def _hash_group(bw_v, base):
    h = jnp.full((L,), FNV_INIT, dtype=jnp.int32)
    for i in range(NGRAM):
        b = bw_v.at[i][pl.ds(base, L)]
        h = (h ^ b) * FNV_PRIME
    uf = h.astype(jnp.float32) + jnp.where(h < 0, TWO32_F, 0.0)
    q = (uf * INV_SLOTS).astype(jnp.int32)
    r = h - q * N_SLOTS
    r = jnp.where(r < 0, r + N_SLOTS, r)
    r = jnp.where(r >= N_SLOTS, r - N_SLOTS, r)
    return r


def _body(
    bw_hbm, bank_hbm, out_hbm,
    bw_v, addr_v, rows_v, stage_v,
    gsem0, gsem1, wsem0, wsem1,
):
    gsems = (gsem0, gsem1)
    wsems = (wsem0, wsem1)
    wid = lax.axis_index("s") * NC + lax.axis_index("c")
    wb = wid // W_PER_B
    ws = (wid % W_PER_B) * TOK_PER_W
    pltpu.sync_copy(bw_hbm.at[wb, :, pl.ds(ws, TOK_PER_W)], bw_v)
    iota = lax.iota(jnp.int32, L)

    def hash_sub(k):
        kb = k % 2

        def group_body(g, _):
            addr = _hash_group(bw_v, k * SUB + g * L)
            addr_v.at[kb][pl.ds(g * L, L)] = addr
            return 0

        lax.fori_loop(0, GROUPS, group_body, 0)

    def gather_desc(k):
        kb = k % 2
        return pltpu.make_async_copy(
            bank_hbm.at[addr_v.at[kb]], rows_v.at[kb], gsems[kb]
        )

    def wb_desc(k):
        kb = k % 2
        s0 = ws + k * SUB
        return pltpu.make_async_copy(
            stage_v.at[kb], out_hbm.at[wb, :, :, pl.ds(s0, SUB)], wsems[kb]
        )

    def repack(k):
        kb = k % 2

        def col_body(c, _):
            col = jnp.full((L,), c, dtype=jnp.int32)
            jv = col // D_STATE
            dv = col % D_STATE
            for g2 in range(GROUPS):
                rows16 = plsc.load_gather(rows_v.at[kb], [g2 * L + iota, col])
                plsc.store_scatter(
                    stage_v.at[kb], [jv, dv, g2 * L + iota], rows16
                )
            return 0

        lax.fori_loop(0, ROW, col_body, 0)

    hash_sub(0)
    gather_desc(0).start()
    for k in range(NSUB):
        if k + 1 < NSUB:
            hash_sub(k + 1)
            gather_desc(k + 1).start()
        gather_desc(k).wait()
        if k >= 2:
            wb_desc(k - 2).wait()
        repack(k)
        wb_desc(k).start()
    wb_desc(NSUB - 2).wait()
    wb_desc(NSUB - 1).wait()


@jax.jit
def _sc_gather(bw_t, bank64):
    mesh = plsc.VectorSubcoreMesh(
        core_axis_name="c", subcore_axis_name="s", num_cores=NC, num_subcores=NS
    )
    return pl.kernel(
        _body,
        out_type=jax.ShapeDtypeStruct(
            (B_WIN, N_BLADES, D_STATE, S_WIN), jnp.float32
        ),
        mesh=mesh,
        scratch_types=[
            pltpu.VMEM((NGRAM, TOK_PER_W), jnp.int32),
            pltpu.VMEM((2, SUB), jnp.int32),
            pltpu.VMEM((2, SUB, ROW), jnp.float32),
            pltpu.VMEM((2, N_BLADES, D_STATE, SUB), jnp.float32),
            pltpu.SemaphoreType.DMA,
            pltpu.SemaphoreType.DMA,
            pltpu.SemaphoreType.DMA,
            pltpu.SemaphoreType.DMA,
        ],
        compiler_params=pltpu.CompilerParams(
            needs_layout_passes=False, use_tc_tiling_on_sc=False
        ),
    )(bw_t, bank64)


def kernel(byte_window, bank):
    bw_t = jnp.transpose(byte_window, (0, 2, 1))
    bank64 = jnp.transpose(bank, (1, 0, 2)).reshape(N_SLOTS, ROW)
    out_t = _sc_gather(bw_t, bank64)
    return jnp.transpose(out_t, (0, 3, 1, 2))


# speedup vs baseline: 1.1595x; 1.0983x over previous
import functools

import jax
import jax.numpy as jnp
from jax import lax
from jax.experimental import pallas as pl
from jax.experimental.pallas import tpu as pltpu
from jax.experimental.pallas import tpu_sc as plsc

N_SLOTS = 100000
D_STATE = 8
NGRAM = 16
N_BLADES = 8
ROW = N_BLADES * D_STATE

NC = 2
NS = 16
L = 16
NW = NC * NS

B_WIN = 16
S_WIN = 4096
TOKENS = B_WIN * S_WIN
TOK_PER_W = TOKENS // NW
SUB = 256
NSUB = TOK_PER_W // SUB
GROUPS = SUB // L
W_PER_B = S_WIN // TOK_PER_W

FNV_INIT = -2128831035
FNV_PRIME = 16777619
TWO32_F = 4294967296.0
INV_SLOTS = 1.0 / float(N_SLOTS)


def _hash_group(bw_v, base):
    h = jnp.full((L,), FNV_INIT, dtype=jnp.int32)
    for i in range(NGRAM):
        b = bw_v.at[i][pl.ds(base, L)]
        h = (h ^ b) * FNV_PRIME
    uf = h.astype(jnp.float32) + jnp.where(h < 0, TWO32_F, 0.0)
    q = (uf * INV_SLOTS).astype(jnp.int32)
    r = h - q * N_SLOTS
    r = jnp.where(r < 0, r + N_SLOTS, r)
    r = jnp.where(r >= N_SLOTS, r - N_SLOTS, r)
    return r


def _body(
    bw_hbm, bank_hbm, out_hbm,
    bw_v, addr_v, rows_v, stage_v,
    gsem0, gsem1, wsem0, wsem1,
):
    gsems = (gsem0, gsem1)
    wsems = (wsem0, wsem1)
    wid = lax.axis_index("s") * NC + lax.axis_index("c")
    wb = wid // W_PER_B
    ws = (wid % W_PER_B) * TOK_PER_W
    pltpu.sync_copy(bw_hbm.at[wb, :, pl.ds(ws, TOK_PER_W)], bw_v)
    iota = lax.iota(jnp.int32, L)

    def hash_sub(k):
        kb = k % 2

        def group_body(g, _):
            addr = _hash_group(bw_v, k * SUB + g * L)
            addr_v.at[kb][pl.ds(g * L, L)] = addr
            return 0

        lax.fori_loop(0, GROUPS, group_body, 0)

    def gather_desc(k):
        kb = k % 2
        return pltpu.make_async_copy(
            bank_hbm.at[addr_v.at[kb]], rows_v.at[kb], gsems[kb]
        )

    def wb_desc(k):
        kb = k % 2
        sc0 = (ws + k * SUB) // 128
        return pltpu.make_async_copy(
            stage_v.at[kb],
            out_hbm.at[wb, :, pl.ds(sc0, SUB // 128), :, :],
            wsems[kb],
        )

    def repack(k):
        kb = k % 2

        def col_body(c, _):
            col = jnp.full((L,), c, dtype=jnp.int32)
            jv = col // D_STATE
            dv = col % D_STATE
            for g2 in range(GROUPS):
                rows16 = plsc.load_gather(rows_v.at[kb], [g2 * L + iota, col])
                sl = g2 * L + iota
                plsc.store_scatter(
                    stage_v.at[kb], [jv, sl // 128, dv, sl % 128], rows16
                )
            return 0

        lax.fori_loop(0, ROW, col_body, 0)

    hash_sub(0)
    gather_desc(0).start()
    for k in range(NSUB):
        if k + 1 < NSUB:
            hash_sub(k + 1)
            gather_desc(k + 1).start()
        gather_desc(k).wait()
        if k >= 2:
            wb_desc(k - 2).wait()
        repack(k)
        wb_desc(k).start()
    wb_desc(NSUB - 2).wait()
    wb_desc(NSUB - 1).wait()


@jax.jit
def _sc_gather(bw_t, bank64):
    mesh = plsc.VectorSubcoreMesh(
        core_axis_name="c", subcore_axis_name="s", num_cores=NC, num_subcores=NS
    )
    return pl.kernel(
        _body,
        out_type=jax.ShapeDtypeStruct(
            (B_WIN, N_BLADES, S_WIN // 128, D_STATE, 128), jnp.float32
        ),
        mesh=mesh,
        scratch_types=[
            pltpu.VMEM((NGRAM, TOK_PER_W), jnp.int32),
            pltpu.VMEM((2, SUB), jnp.int32),
            pltpu.VMEM((2, SUB, ROW), jnp.float32),
            pltpu.VMEM((2, N_BLADES, SUB // 128, D_STATE, 128), jnp.float32),
            pltpu.SemaphoreType.DMA,
            pltpu.SemaphoreType.DMA,
            pltpu.SemaphoreType.DMA,
            pltpu.SemaphoreType.DMA,
        ],
        compiler_params=pltpu.CompilerParams(
            needs_layout_passes=False, use_tc_tiling_on_sc=False
        ),
    )(bw_t, bank64)


def kernel(byte_window, bank):
    bw_t = jnp.transpose(byte_window, (0, 2, 1))
    bank64 = jnp.transpose(bank, (1, 0, 2)).reshape(N_SLOTS, ROW)
    out5 = _sc_gather(bw_t, bank64)
    out_t = jnp.transpose(out5, (0, 1, 3, 2, 4)).reshape(
        B_WIN, N_BLADES, D_STATE, S_WIN
    )
    return jnp.transpose(out_t, (0, 3, 1, 2))
